# Initial kernel scaffold; baseline (speedup 1.0000x reference)
import jax
import jax.numpy as jnp
from jax.experimental import pallas as pl


def kernel(x, table, W, b):
    raise NotImplementedError("write your pallas kernel here")


# speedup vs baseline: 50.7249x; 50.7249x over previous
import functools

import jax
import jax.numpy as jnp
from jax import lax
from jax.experimental import pallas as pl
from jax.experimental.pallas import tpu as pltpu
from jax.experimental.pallas import tpu_sc as plsc

SEQ = 200
BATCH = 4096
VOCAB = 25006
EMB = 100
VPAD = 25600

_NC = 2
_NS = 16
_NW = _NC * _NS
_BPW = BATCH // _NW
_L = 16
_JG = _BPW // _L


def _proj_body(tab_ref, w_ref, out_ref):
    rowsum = jnp.sum(tab_ref[...] * w_ref[...], axis=1) * (1.0 / SEQ)
    out_ref[...] = rowsum.reshape(8, 128)


def _project_table(table, W):
    t2d = pl.pallas_call(
        _proj_body,
        grid=(VPAD // 1024,),
        in_specs=[
            pl.BlockSpec((1024, EMB), lambda i: (i, 0)),
            pl.BlockSpec((1, EMB), lambda i: (0, 0)),
        ],
        out_specs=pl.BlockSpec((8, 128), lambda i: (i, 0)),
        out_shape=jax.ShapeDtypeStruct((VPAD // 128, 128), jnp.float32),
    )(table, W)
    return t2d.reshape(VPAD)


def _sc_body(x_hbm, t_hbm, bias_hbm, out_hbm, x_v, t_v, bias_v, acc_v):
    wid = lax.axis_index("s") * _NC + lax.axis_index("c")
    base = wid * _BPW
    pltpu.sync_copy(t_hbm, t_v)
    pltpu.sync_copy(x_hbm.at[:, pl.ds(base, _BPW)], x_v)
    pltpu.sync_copy(bias_hbm, bias_v)

    def step(s, accs):
        new = []
        for j in range(_JG):
            idx = x_v[s, pl.ds(j * _L, _L)]
            new.append(accs[j] + plsc.load_gather(t_v, [idx]))
        return tuple(new)

    zero = jnp.zeros((_L,), jnp.float32)
    accs = lax.fori_loop(0, SEQ, step, (zero,) * _JG)
    bias = bias_v[...]
    for j in range(_JG):
        acc_v[pl.ds(j * _L, _L)] = accs[j] + bias
    pltpu.sync_copy(acc_v, out_hbm.at[pl.ds(base, _BPW)])


@functools.partial(jax.jit, static_argnames=())
def kernel(x, table, W, b):
    t = _project_table(table, W)
    bias16 = jnp.broadcast_to(b, (_L,))
    sc = pl.kernel(
        _sc_body,
        out_type=jax.ShapeDtypeStruct((BATCH,), jnp.float32),
        mesh=plsc.VectorSubcoreMesh(core_axis_name="c", subcore_axis_name="s"),
        scratch_types=[
            pltpu.VMEM((SEQ, _BPW), jnp.int32),
            pltpu.VMEM((VPAD,), jnp.float32),
            pltpu.VMEM((_L,), jnp.float32),
            pltpu.VMEM((_BPW,), jnp.float32),
        ],
        compiler_params=pltpu.CompilerParams(needs_layout_passes=False),
    )
    out = sc(x, t, bias16)
    return out.reshape(BATCH, 1, 1)


# speedup vs baseline: 56.4660x; 1.1132x over previous
import functools

import jax
import jax.numpy as jnp
from jax import lax
from jax.experimental import pallas as pl
from jax.experimental.pallas import tpu as pltpu
from jax.experimental.pallas import tpu_sc as plsc

SEQ = 200
BATCH = 4096
VOCAB = 25006
EMB = 100
VPAD = 26624

_NC = 2
_NS = 16
_NW = _NC * _NS
_BPW = BATCH // _NW
_L = 16
_JG = _BPW // _L


def _proj_body(tab_ref, w_ref, b_ref, out_ref, bias_ref):
    rowsum = jnp.sum(tab_ref[...] * w_ref[...], axis=1) * (1.0 / SEQ)
    out_ref[...] = rowsum.reshape(16, 128)
    bias_ref[...] = jnp.full((_L,), b_ref[0], jnp.float32)


def _project_table(table, W, b):
    t2d, bias16 = pl.pallas_call(
        _proj_body,
        grid=(VPAD // 2048,),
        in_specs=[
            pl.BlockSpec((2048, EMB), lambda i: (i, 0)),
            pl.BlockSpec((1, EMB), lambda i: (0, 0)),
            pl.BlockSpec((1,), lambda i: (0,)),
        ],
        out_specs=[
            pl.BlockSpec((16, 128), lambda i: (i, 0)),
            pl.BlockSpec((_L,), lambda i: (0,)),
        ],
        out_shape=[
            jax.ShapeDtypeStruct((VPAD // 128, 128), jnp.float32),
            jax.ShapeDtypeStruct((_L,), jnp.float32),
        ],
    )(table, W, b)
    return t2d.reshape(VPAD), bias16


def _sc_body(x_hbm, t_hbm, bias_hbm, out_hbm, x_v, t_v, bias_v, acc_v):
    wid = lax.axis_index("s") * _NC + lax.axis_index("c")
    base = wid * _BPW
    pltpu.sync_copy(t_hbm, t_v)
    pltpu.sync_copy(x_hbm.at[:, pl.ds(base, _BPW)], x_v)
    pltpu.sync_copy(bias_hbm, bias_v)

    def step(s, accs):
        new = []
        for j in range(_JG):
            idx = x_v[s, pl.ds(j * _L, _L)]
            new.append(accs[j] + plsc.load_gather(t_v, [idx]))
        return tuple(new)

    zero = jnp.zeros((_L,), jnp.float32)
    accs = lax.fori_loop(0, SEQ, step, (zero,) * _JG)
    bias = bias_v[...]
    for j in range(_JG):
        acc_v[pl.ds(j * _L, _L)] = accs[j] + bias
    pltpu.sync_copy(acc_v, out_hbm.at[pl.ds(base, _BPW)])


@functools.partial(jax.jit, static_argnames=())
def kernel(x, table, W, b):
    t, bias16 = _project_table(table, W, b)
    sc = pl.kernel(
        _sc_body,
        out_type=jax.ShapeDtypeStruct((BATCH,), jnp.float32),
        mesh=plsc.VectorSubcoreMesh(core_axis_name="c", subcore_axis_name="s"),
        scratch_types=[
            pltpu.VMEM((SEQ, _BPW), jnp.int32),
            pltpu.VMEM((VPAD,), jnp.float32),
            pltpu.VMEM((_L,), jnp.float32),
            pltpu.VMEM((_BPW,), jnp.float32),
        ],
        compiler_params=pltpu.CompilerParams(needs_layout_passes=False),
    )
    out = sc(x, t, bias16)
    return out.reshape(BATCH, 1, 1)


# speedup vs baseline: 75.7364x; 1.3413x over previous
import functools

import jax
import jax.numpy as jnp
from jax import lax
from jax.experimental import pallas as pl
from jax.experimental.pallas import tpu as pltpu
from jax.experimental.pallas import tpu_sc as plsc

SEQ = 200
BATCH = 4096
VOCAB = 25006
EMB = 100
VPAD = 26624

_NC = 2
_NS = 16
_NW = _NC * _NS
_BPW = BATCH // _NW
_L = 16
_JG = _BPW // _L


def _proj_body(tabT_ref, wT_ref, b_ref, out_ref, bias_ref):
    colsum = jnp.sum(tabT_ref[...] * wT_ref[...], axis=0) * (1.0 / SEQ)
    out_ref[...] = colsum.reshape(16, 128)
    bias_ref[...] = jnp.full((_L,), b_ref[0], jnp.float32)


def _project_table(tableT, WT, b):
    t2d, bias16 = pl.pallas_call(
        _proj_body,
        grid=(VPAD // 2048,),
        in_specs=[
            pl.BlockSpec((EMB, 2048), lambda i: (0, i)),
            pl.BlockSpec((EMB, 1), lambda i: (0, 0)),
            pl.BlockSpec((1,), lambda i: (0,)),
        ],
        out_specs=[
            pl.BlockSpec((16, 128), lambda i: (i, 0)),
            pl.BlockSpec((_L,), lambda i: (0,)),
        ],
        out_shape=[
            jax.ShapeDtypeStruct((VPAD // 128, 128), jnp.float32),
            jax.ShapeDtypeStruct((_L,), jnp.float32),
        ],
    )(tableT, WT, b)
    return t2d.reshape(VPAD), bias16


def _sc_body(x_hbm, t_hbm, bias_hbm, out_hbm, x_v, t_v, bias_v, acc_v):
    wid = lax.axis_index("s") * _NC + lax.axis_index("c")
    base = wid * _BPW
    pltpu.sync_copy(t_hbm, t_v)
    pltpu.sync_copy(x_hbm.at[:, pl.ds(base, _BPW)], x_v)
    pltpu.sync_copy(bias_hbm, bias_v)

    def step(s, accs):
        new = []
        for j in range(_JG):
            idx = x_v[s, pl.ds(j * _L, _L)]
            new.append(accs[j] + plsc.load_gather(t_v, [idx]))
        return tuple(new)

    zero = jnp.zeros((_L,), jnp.float32)
    accs = lax.fori_loop(0, SEQ, step, (zero,) * _JG)
    bias = bias_v[...]
    for j in range(_JG):
        acc_v[pl.ds(j * _L, _L)] = accs[j] + bias
    pltpu.sync_copy(acc_v, out_hbm.at[pl.ds(base, _BPW)])


@functools.partial(jax.jit, static_argnames=())
def kernel(x, table, W, b):
    t, bias16 = _project_table(table.T, W.T, b)
    sc = pl.kernel(
        _sc_body,
        out_type=jax.ShapeDtypeStruct((BATCH,), jnp.float32),
        mesh=plsc.VectorSubcoreMesh(core_axis_name="c", subcore_axis_name="s"),
        scratch_types=[
            pltpu.VMEM((SEQ, _BPW), jnp.int32),
            pltpu.VMEM((VPAD,), jnp.float32),
            pltpu.VMEM((_L,), jnp.float32),
            pltpu.VMEM((_BPW,), jnp.float32),
        ],
        compiler_params=pltpu.CompilerParams(needs_layout_passes=False),
    )
    out = sc(x, t, bias16)
    return out.reshape(BATCH, 1, 1)


# speedup vs baseline: 80.8448x; 1.0674x over previous
import functools

import jax
import jax.numpy as jnp
from jax import lax
from jax.experimental import pallas as pl
from jax.experimental.pallas import tpu as pltpu
from jax.experimental.pallas import tpu_sc as plsc

SEQ = 200
BATCH = 4096
VOCAB = 25006
EMB = 100
VPAD = 26624

_NC = 2
_NS = 16
_NW = _NC * _NS
_BPW = BATCH // _NW
_L = 16
_JG = _BPW // _L


def _proj_body(tabT_ref, w_ref, b_ref, out_ref, bias_ref):
    w_col = jnp.transpose(w_ref[...])
    colsum = jnp.sum(tabT_ref[...] * w_col, axis=0) * (1.0 / SEQ)
    out_ref[...] = colsum.reshape(16, 128)
    bias_ref[...] = jnp.full((_L,), b_ref[0], jnp.float32)


def _project_table(tableT, W, b):
    t2d, bias16 = pl.pallas_call(
        _proj_body,
        grid=(VPAD // 2048,),
        in_specs=[
            pl.BlockSpec((EMB, 2048), lambda i: (0, i)),
            pl.BlockSpec((1, EMB), lambda i: (0, 0)),
            pl.BlockSpec((1,), lambda i: (0,)),
        ],
        out_specs=[
            pl.BlockSpec((16, 128), lambda i: (i, 0)),
            pl.BlockSpec((_L,), lambda i: (0,)),
        ],
        out_shape=[
            jax.ShapeDtypeStruct((VPAD // 128, 128), jnp.float32),
            jax.ShapeDtypeStruct((_L,), jnp.float32),
        ],
    )(tableT, W, b)
    return t2d.reshape(VPAD), bias16


def _sc_body(x_hbm, t_hbm, bias_hbm, out_hbm, x_v, t_v, bias_v, acc_v,
             sem_t, sem_x, sem_b):
    wid = lax.axis_index("s") * _NC + lax.axis_index("c")
    base = wid * _BPW
    cp_t = pltpu.make_async_copy(t_hbm, t_v, sem_t)
    cp_x = pltpu.make_async_copy(x_hbm.at[:, pl.ds(base, _BPW)], x_v, sem_x)
    cp_b = pltpu.make_async_copy(bias_hbm, bias_v, sem_b)
    cp_t.start()
    cp_x.start()
    cp_b.start()
    cp_t.wait()
    cp_x.wait()

    zero = jnp.zeros((_L,), jnp.float32)

    @plsc.parallel_loop(0, SEQ, 1, unroll=4, carry=(zero,) * _JG)
    def accs(s, accs_in):
        new = []
        for j in range(_JG):
            idx = x_v[s, pl.ds(j * _L, _L)]
            new.append(accs_in[j] + plsc.load_gather(t_v, [idx]))
        return tuple(new)

    cp_b.wait()
    bias = bias_v[...]
    for j in range(_JG):
        acc_v[pl.ds(j * _L, _L)] = accs[j] + bias
    pltpu.sync_copy(acc_v, out_hbm.at[pl.ds(base, _BPW)])


@functools.partial(jax.jit, static_argnames=())
def kernel(x, table, W, b):
    t, bias16 = _project_table(table.T, W, b)
    sc = pl.kernel(
        _sc_body,
        out_type=jax.ShapeDtypeStruct((BATCH,), jnp.float32),
        mesh=plsc.VectorSubcoreMesh(core_axis_name="c", subcore_axis_name="s"),
        scratch_types=[
            pltpu.VMEM((SEQ, _BPW), jnp.int32),
            pltpu.VMEM((VPAD,), jnp.float32),
            pltpu.VMEM((_L,), jnp.float32),
            pltpu.VMEM((_BPW,), jnp.float32),
            pltpu.SemaphoreType.DMA,
            pltpu.SemaphoreType.DMA,
            pltpu.SemaphoreType.DMA,
        ],
        compiler_params=pltpu.CompilerParams(needs_layout_passes=False),
    )
    out = sc(x, t, bias16)
    return out.reshape(BATCH, 1, 1)


# speedup vs baseline: 94.3858x; 1.1675x over previous
import functools

import jax
import jax.numpy as jnp
from jax import lax
from jax.experimental import pallas as pl
from jax.experimental.pallas import tpu as pltpu
from jax.experimental.pallas import tpu_sc as plsc

SEQ = 200
BATCH = 4096
VOCAB = 25006
EMB = 100
VPAD = 26624

_NC = 2
_NS = 16
_NW = _NC * _NS
_BPW = BATCH // _NW
_L = 16
_JG = _BPW // _L


def _proj_body(tabT_ref, w_ref, b_ref, out_ref, bias_ref):
    w_col = jnp.transpose(w_ref[...])
    colsum = jnp.sum(tabT_ref[...] * w_col, axis=0) * (1.0 / SEQ)
    out_ref[...] = colsum.reshape(104, 128)
    bias_ref[...] = jnp.full((_L,), b_ref[0], jnp.float32)


def _project_table(tableT, W, b):
    t2d, bias16 = pl.pallas_call(
        _proj_body,
        grid=(VPAD // 13312,),
        in_specs=[
            pl.BlockSpec((EMB, 13312), lambda i: (0, i)),
            pl.BlockSpec((1, EMB), lambda i: (0, 0)),
            pl.BlockSpec((1,), lambda i: (0,)),
        ],
        out_specs=[
            pl.BlockSpec((104, 128), lambda i: (i, 0)),
            pl.BlockSpec((_L,), lambda i: (0,)),
        ],
        out_shape=[
            jax.ShapeDtypeStruct((VPAD // 128, 128), jnp.float32),
            jax.ShapeDtypeStruct((_L,), jnp.float32),
        ],
    )(tableT, W, b)
    return t2d.reshape(VPAD), bias16


def _sc_body(x_hbm, t_hbm, bias_hbm, out_hbm, x_v, t_v, bias_v, acc_v,
             sem_t, sem_x, sem_b):
    wid = lax.axis_index("s") * _NC + lax.axis_index("c")
    base = wid * _BPW
    cp_t = pltpu.make_async_copy(t_hbm, t_v, sem_t)
    cp_x = pltpu.make_async_copy(x_hbm.at[:, pl.ds(base, _BPW)], x_v, sem_x)
    cp_b = pltpu.make_async_copy(bias_hbm, bias_v, sem_b)
    cp_t.start()
    cp_x.start()
    cp_b.start()
    cp_t.wait()
    cp_x.wait()

    zero = jnp.zeros((_L,), jnp.float32)

    @plsc.parallel_loop(0, SEQ, 1, unroll=4, carry=(zero,) * _JG)
    def accs(s, accs_in):
        new = []
        for j in range(_JG):
            idx = x_v[s, pl.ds(j * _L, _L)]
            new.append(accs_in[j] + plsc.load_gather(t_v, [idx]))
        return tuple(new)

    cp_b.wait()
    bias = bias_v[...]
    for j in range(_JG):
        acc_v[pl.ds(j * _L, _L)] = accs[j] + bias
    pltpu.sync_copy(acc_v, out_hbm.at[pl.ds(base, _BPW)])


@functools.partial(jax.jit, static_argnames=())
def kernel(x, table, W, b):
    t, bias16 = _project_table(table.T, W, b)
    sc = pl.kernel(
        _sc_body,
        out_type=jax.ShapeDtypeStruct((BATCH,), jnp.float32),
        mesh=plsc.VectorSubcoreMesh(core_axis_name="c", subcore_axis_name="s"),
        scratch_types=[
            pltpu.VMEM((SEQ, _BPW), jnp.int32),
            pltpu.VMEM((VPAD,), jnp.float32),
            pltpu.VMEM((_L,), jnp.float32),
            pltpu.VMEM((_BPW,), jnp.float32),
            pltpu.SemaphoreType.DMA,
            pltpu.SemaphoreType.DMA,
            pltpu.SemaphoreType.DMA,
        ],
        compiler_params=pltpu.CompilerParams(needs_layout_passes=False),
    )
    out = sc(x, t, bias16)
    return out.reshape(BATCH, 1, 1)
